# baseline (speedup 1.0000x reference)
import functools

import jax
import jax.numpy as jnp
from jax import lax
from jax.experimental import pallas as pl
from jax.experimental.pallas import tpu as pltpu
from jax.experimental.pallas import tpu_sc as plsc

_NC = 2
_NS = 16
_NW = _NC * _NS


def _sc_gather(user_ids, item_ids, user_emb, item_emb, chunk_base, chunk_rows):
    D = user_emb.shape[1]
    b_per_w = chunk_rows // _NW
    mesh = plsc.VectorSubcoreMesh(
        core_axis_name="c", subcore_axis_name="s", num_cores=_NC, num_subcores=_NS
    )

    @functools.partial(
        pl.kernel,
        mesh=mesh,
        out_type=(
            jax.ShapeDtypeStruct((chunk_rows, D), jnp.float32),
            jax.ShapeDtypeStruct((chunk_rows, D), jnp.float32),
        ),
        scratch_types=[
            pltpu.VMEM((b_per_w,), jnp.int32),
            pltpu.VMEM((b_per_w,), jnp.int32),
            pltpu.VMEM((b_per_w, D), jnp.float32),
            pltpu.VMEM((b_per_w, D), jnp.float32),
            pltpu.SemaphoreType.DMA,
            pltpu.SemaphoreType.DMA,
            pltpu.SemaphoreType.DMA,
            pltpu.SemaphoreType.DMA,
        ],
    )
    def gather_kernel(uid_hbm, iid_hbm, uemb_hbm, iemb_hbm, uout_hbm, iout_hbm,
                      idx_u, idx_i, rows_u, rows_i, gs_u, gs_i, ss_u, ss_i):
        wid = lax.axis_index("s") * _NC + lax.axis_index("c")
        src_base = chunk_base + wid * b_per_w
        dst_base = wid * b_per_w
        pltpu.sync_copy(uid_hbm.at[pl.ds(src_base, b_per_w)], idx_u)
        pltpu.sync_copy(iid_hbm.at[pl.ds(src_base, b_per_w)], idx_i)
        gu = pltpu.async_copy(uemb_hbm.at[idx_u], rows_u, gs_u)
        gi = pltpu.async_copy(iemb_hbm.at[idx_i], rows_i, gs_i)
        gu.wait()
        su = pltpu.async_copy(rows_u, uout_hbm.at[pl.ds(dst_base, b_per_w)], ss_u)
        gi.wait()
        si = pltpu.async_copy(rows_i, iout_hbm.at[pl.ds(dst_base, b_per_w)], ss_i)
        su.wait()
        si.wait()

    return gather_kernel(user_ids, item_ids, user_emb, item_emb)


def _mlp_body(u_ref, i_ref, w1_ref, b1_ref, w2_ref, b2_ref,
              w3_ref, b3_ref, w4_ref, b4_ref, out_ref):
    bf = jnp.bfloat16
    f32 = jnp.float32
    xc = jnp.concatenate([u_ref[...], i_ref[...]], axis=1)
    x = jnp.dot(xc.astype(bf), w1_ref[...],
                preferred_element_type=f32).astype(bf)
    h = jnp.maximum(x + b1_ref[...], 0)
    enc = (jnp.dot(h, w2_ref[...], preferred_element_type=f32).astype(bf)
           + b2_ref[...])
    h2 = jnp.maximum(
        jnp.dot(enc, w3_ref[...], preferred_element_type=f32).astype(bf)
        + b3_ref[...], 0
    )
    prod = h2.astype(f32) * w4_ref[...]
    out_ref[...] = jnp.sum(prod, axis=1) + b4_ref[0, 0]


def _mlp(u, i, W1, b1, W2, b2, W3, b3, w4row, b4, tile_m=4096):
    B, D = u.shape
    D2 = 2 * D
    H = W1.shape[1]
    L = W2.shape[1]
    grid = (B // tile_m,)
    fixed = lambda m: (0, 0)
    out = pl.pallas_call(
        _mlp_body,
        grid=grid,
        in_specs=[
            pl.BlockSpec((tile_m, D), lambda m: (m, 0)),
            pl.BlockSpec((tile_m, D), lambda m: (m, 0)),
            pl.BlockSpec((D2, H), fixed),
            pl.BlockSpec((1, H), fixed),
            pl.BlockSpec((H, L), fixed),
            pl.BlockSpec((1, L), fixed),
            pl.BlockSpec((L, H), fixed),
            pl.BlockSpec((1, H), fixed),
            pl.BlockSpec((1, H), fixed),
            pl.BlockSpec((1, 1), fixed),
        ],
        out_specs=pl.BlockSpec((tile_m,), lambda m: (m,)),
        out_shape=jax.ShapeDtypeStruct((B,), jnp.float32),
    )(u, i, W1, b1, W2, b2, W3, b3, w4row, b4)
    return out


def kernel(user_ids, item_ids, user_emb, item_emb, W1, b1, W2, b2, W3, b3, W4, b4):
    B = user_ids.shape[0]
    bf = jnp.bfloat16
    nchunks = 2
    C = B // nchunks
    mlp_args = (
        W1.astype(bf),
        b1.reshape(1, -1).astype(bf), W2.astype(bf), b2.reshape(1, -1).astype(bf),
        W3.astype(bf), b3.reshape(1, -1).astype(bf),
        W4.reshape(1, -1), b4.reshape(1, 1),
    )
    xs = [
        _sc_gather(user_ids, item_ids, user_emb, item_emb, c * C, C)
        for c in range(nchunks)
    ]
    outs = [_mlp(u, i, *mlp_args) for (u, i) in xs]
    return jnp.concatenate(outs, axis=0)


# speedup vs baseline: 1.2004x; 1.2004x over previous
import functools

import jax
import jax.numpy as jnp
from jax import lax
from jax.experimental import pallas as pl
from jax.experimental.pallas import tpu as pltpu
from jax.experimental.pallas import tpu_sc as plsc

_NC = 2
_NS = 16
_NW = _NC * _NS


def _sc_gather(user_ids, item_ids, user_emb, item_emb, chunk_base, chunk_rows):
    D = user_emb.shape[1]
    b_per_w = chunk_rows // _NW
    mesh = plsc.VectorSubcoreMesh(
        core_axis_name="c", subcore_axis_name="s", num_cores=_NC, num_subcores=_NS
    )

    @functools.partial(
        pl.kernel,
        mesh=mesh,
        out_type=(
            jax.ShapeDtypeStruct((chunk_rows, D), jnp.float32),
            jax.ShapeDtypeStruct((chunk_rows, D), jnp.float32),
        ),
        scratch_types=[
            pltpu.VMEM((b_per_w,), jnp.int32),
            pltpu.VMEM((b_per_w,), jnp.int32),
            pltpu.VMEM((b_per_w, D), jnp.float32),
            pltpu.VMEM((b_per_w, D), jnp.float32),
            pltpu.SemaphoreType.DMA,
            pltpu.SemaphoreType.DMA,
            pltpu.SemaphoreType.DMA,
            pltpu.SemaphoreType.DMA,
        ],
    )
    def gather_kernel(uid_hbm, iid_hbm, uemb_hbm, iemb_hbm, uout_hbm, iout_hbm,
                      idx_u, idx_i, rows_u, rows_i, gs_u, gs_i, ss_u, ss_i):
        wid = lax.axis_index("s") * _NC + lax.axis_index("c")
        src_base = chunk_base + wid * b_per_w
        dst_base = wid * b_per_w
        pltpu.sync_copy(uid_hbm.at[pl.ds(src_base, b_per_w)], idx_u)
        pltpu.sync_copy(iid_hbm.at[pl.ds(src_base, b_per_w)], idx_i)
        gu = pltpu.async_copy(uemb_hbm.at[idx_u], rows_u, gs_u)
        gi = pltpu.async_copy(iemb_hbm.at[idx_i], rows_i, gs_i)
        gu.wait()
        su = pltpu.async_copy(rows_u, uout_hbm.at[pl.ds(dst_base, b_per_w)], ss_u)
        gi.wait()
        si = pltpu.async_copy(rows_i, iout_hbm.at[pl.ds(dst_base, b_per_w)], ss_i)
        su.wait()
        si.wait()

    return gather_kernel(user_ids, item_ids, user_emb, item_emb)


def _mlp_body(u_ref, i_ref, w1_ref, b1_ref, w2_ref, b2_ref,
              w3_ref, b3_ref, w4_ref, b4_ref, out_ref):
    bf = jnp.bfloat16
    f32 = jnp.float32
    xc = jnp.concatenate([u_ref[...], i_ref[...]], axis=1)
    x = jnp.dot(xc.astype(bf), w1_ref[...],
                preferred_element_type=f32).astype(bf)
    h = jnp.maximum(x + b1_ref[...], 0)
    enc = (jnp.dot(h, w2_ref[...], preferred_element_type=f32).astype(bf)
           + b2_ref[...])
    h2 = jnp.maximum(
        jnp.dot(enc, w3_ref[...], preferred_element_type=f32).astype(bf)
        + b3_ref[...], 0
    )
    prod = h2.astype(f32) * w4_ref[...]
    out_ref[...] = jnp.sum(prod, axis=1, keepdims=True) + b4_ref[...]


def _mlp(u, i, W1, b1, W2, b2, W3, b3, w4row, b4, tile_m=4096):
    B, D = u.shape
    D2 = 2 * D
    H = W1.shape[1]
    L = W2.shape[1]
    grid = (B // tile_m,)
    fixed = lambda m: (0, 0)
    out = pl.pallas_call(
        _mlp_body,
        grid=grid,
        in_specs=[
            pl.BlockSpec((tile_m, D), lambda m: (m, 0)),
            pl.BlockSpec((tile_m, D), lambda m: (m, 0)),
            pl.BlockSpec((D2, H), fixed),
            pl.BlockSpec((1, H), fixed),
            pl.BlockSpec((H, L), fixed),
            pl.BlockSpec((1, L), fixed),
            pl.BlockSpec((L, H), fixed),
            pl.BlockSpec((1, H), fixed),
            pl.BlockSpec((1, H), fixed),
            pl.BlockSpec((1, 1), fixed),
        ],
        out_specs=pl.BlockSpec((tile_m, 1), lambda m: (m, 0)),
        out_shape=jax.ShapeDtypeStruct((B, 1), jnp.float32),
    )(u, i, W1, b1, W2, b2, W3, b3, w4row, b4)
    return out


def kernel(user_ids, item_ids, user_emb, item_emb, W1, b1, W2, b2, W3, b3, W4, b4):
    B = user_ids.shape[0]
    bf = jnp.bfloat16
    nchunks = 2
    C = B // nchunks
    mlp_args = (
        W1.astype(bf),
        b1.reshape(1, -1).astype(bf), W2.astype(bf), b2.reshape(1, -1).astype(bf),
        W3.astype(bf), b3.reshape(1, -1).astype(bf),
        W4.reshape(1, -1), b4.reshape(1, 1),
    )
    xs = [
        _sc_gather(user_ids, item_ids, user_emb, item_emb, c * C, C)
        for c in range(nchunks)
    ]
    outs = [_mlp(u, i, *mlp_args) for (u, i) in xs]
    return jnp.reshape(jnp.concatenate(outs, axis=0), (B,))
